# baseline (speedup 1.0000x reference)
import functools

import jax
import jax.numpy as jnp
from jax.experimental import pallas as pl
from jax.experimental.pallas import tpu as pltpu


def _mlp2_body(x_ref, a_ref, w0_ref, w1_ref, w2_ref, out_ref):
    h = x_ref[...] + jnp.dot(a_ref[...], w0_ref[...],
                             preferred_element_type=jnp.float32)
    h = jnp.maximum(jnp.dot(h, w1_ref[...], preferred_element_type=jnp.float32), 0.0)
    out_ref[...] = jnp.dot(h, w2_ref[...], preferred_element_type=jnp.float32)


def _mlp2(x, a, w0, w1, w2, block):
    n, d = x.shape
    grid = (n // block,)
    bs = pl.BlockSpec((block, d), lambda i: (i, 0))
    ws = pl.BlockSpec((d, d), lambda i: (0, 0))
    return pl.pallas_call(
        _mlp2_body,
        grid=grid,
        in_specs=[bs, bs, ws, ws, ws],
        out_specs=bs,
        out_shape=jax.ShapeDtypeStruct((n, d), jnp.float32),
    )(x, a, w0, w1, w2)


def _gin_body(x_ref, w1_ref, w2_ref, out_ref):
    h = jnp.maximum(jnp.dot(x_ref[...], w1_ref[...],
                            preferred_element_type=jnp.float32), 0.0)
    out_ref[...] = jnp.dot(h, w2_ref[...], preferred_element_type=jnp.float32)


def _gin(x, w1, w2, block):
    n, d = x.shape
    grid = (n // block,)
    bs = pl.BlockSpec((block, d), lambda i: (i, 0))
    ws = pl.BlockSpec((d, d), lambda i: (0, 0))
    return pl.pallas_call(
        _gin_body,
        grid=grid,
        in_specs=[bs, ws, ws],
        out_specs=bs,
        out_shape=jax.ShapeDtypeStruct((n, d), jnp.float32),
    )(x, w1, w2)


def _edge_body(edge_ref, a1_ref, s_ref, wec1_ref, wec2_ref, wm1a_ref,
               wm1b_ref, wm2_ref, out_ref):
    e1 = jnp.maximum(a1_ref[...], 0.0)
    t = jnp.dot(edge_ref[...], wec1_ref[...],
                preferred_element_type=jnp.float32) + s_ref[...]
    e2 = jnp.dot(jnp.maximum(t, 0.0), wec2_ref[...],
                 preferred_element_type=jnp.float32)
    m = jnp.dot(e1, wm1a_ref[...], preferred_element_type=jnp.float32)
    m += jnp.dot(e2, wm1b_ref[...], preferred_element_type=jnp.float32)
    out_ref[...] = jnp.dot(jnp.maximum(m, 0.0), wm2_ref[...],
                           preferred_element_type=jnp.float32)


def _edge_mlp(edge_rep, a1, s, wec1s, wec2, wm1a, wm1b, wm2, block):
    e, d = edge_rep.shape
    grid = (e // block,)
    bs = pl.BlockSpec((block, d), lambda i: (i, 0))
    ws = pl.BlockSpec((d, d), lambda i: (0, 0))
    return pl.pallas_call(
        _edge_body,
        grid=grid,
        in_specs=[bs, bs, bs, ws, ws, ws, ws, ws],
        out_specs=bs,
        out_shape=jax.ShapeDtypeStruct((e, d), jnp.float32),
    )(edge_rep, a1, s, wec1s, wec2, wm1a, wm1b, wm2)


def kernel(node_rep, edge_rep, cycle_rep, edge_index, c2e_edge, c2e_cycle,
           W_e2n, W_n1, W_n2, W_ne1, W_ec1, W_ec2, W_ce1, W_ce2, W_m1, W_m2,
           eps_e, eps_c):
    n_nodes = node_rep.shape[0]
    n_cycles = cycle_rep.shape[0]
    src = edge_index[0]
    dst = edge_index[1]

    agg_n = jax.ops.segment_sum(edge_rep, dst, num_segments=n_nodes)
    agg_e2c = jax.ops.segment_sum(edge_rep[c2e_edge], c2e_cycle,
                                  num_segments=n_cycles)
    W_a, W_b = W_ne1[:128], W_ne1[128:]
    pa = node_rep @ W_a
    pb = node_rep @ W_b
    a1 = pa[src] + pb[dst]
    cw = cycle_rep @ W_ec1
    s = jnp.zeros(edge_rep.shape, jnp.float32).at[c2e_edge].add(cw[c2e_cycle])

    node_out = _mlp2(node_rep, agg_n, W_e2n, W_n1, W_n2, block=1000)
    cin = (1.0 + eps_c) * cycle_rep + agg_e2c
    cycle_out = _gin(cin, W_ce1, W_ce2, block=1024)
    wec1s = (1.0 + eps_e) * W_ec1
    W_m1a, W_m1b = W_m1[:128], W_m1[128:]
    edge_out = _edge_mlp(edge_rep, a1, s, wec1s, W_ec2, W_m1a, W_m1b, W_m2,
                         block=2000)
    return (node_out, edge_out, cycle_out)


# speedup vs baseline: 1.0537x; 1.0537x over previous
import functools

import jax
import jax.numpy as jnp
from jax import lax
from jax.experimental import pallas as pl
from jax.experimental.pallas import tpu as pltpu
from jax.experimental.pallas import tpu_sc as plsc

_NC = 2
_NS = 16
_L = 16


def _mesh():
    return plsc.VectorSubcoreMesh(core_axis_name="c", subcore_axis_name="s")


def _zero_zbuf(zbuf):
    def zrow(i, _):
        for u in range(zbuf.shape[1] // _L):
            zbuf[i, pl.ds(u * _L, _L)] = jnp.zeros((_L,), jnp.float32)
        return 0
    lax.fori_loop(0, zbuf.shape[0], zrow, 0)


def _zero_acc_range(acc, zbuf, start, n):
    z = zbuf.shape[0]
    n_full, rem = n // z, n % z
    def zacc(i, _):
        pltpu.sync_copy(zbuf, acc.at[pl.ds(start + i * z, z)])
        return 0
    lax.fori_loop(0, n_full, zacc, 0)
    if rem:
        pltpu.sync_copy(zbuf.at[pl.ds(0, rem)],
                        acc.at[pl.ds(start + n_full * z, rem)])


def _repack(src1d, dst2d):
    nch, ch = dst2d.shape
    def rp(j, _):
        for u in range(ch // _L):
            dst2d[j, pl.ds(u * _L, _L)] = src1d[pl.ds(j * ch + u * _L, _L)]
        return 0
    lax.fori_loop(0, nch, rp, 0)


def _sc_agg_n(edge_rep, dst, n_nodes):
    E, D = edge_rep.shape
    EPT = E // (_NC * _NS)
    CH = 80
    NCH = EPT // CH
    NPAD = -(-n_nodes // 128) * 128
    ZPT = NPAD // _NS

    @functools.partial(
        pl.kernel,
        out_type=jax.ShapeDtypeStruct((_NC * NPAD, D), jnp.float32),
        mesh=_mesh(),
        scratch_types=[
            pltpu.VMEM_SHARED((NPAD, D), jnp.float32),
            pltpu.VMEM((EPT,), jnp.int32),
            pltpu.VMEM((NCH, CH), jnp.int32),
            pltpu.VMEM((CH, D), jnp.float32),
            pltpu.VMEM((32, D), jnp.float32),
        ],
    )
    def k(edge_hbm, dst_hbm, out_hbm, acc, dbuf, d2, rows, zbuf):
        c = lax.axis_index("c")
        s = lax.axis_index("s")
        _zero_zbuf(zbuf)
        _zero_acc_range(acc, zbuf, s * ZPT, ZPT)
        tbase = (c * _NS + s) * EPT
        pltpu.sync_copy(dst_hbm.at[pl.ds(tbase, EPT)], dbuf)
        _repack(dbuf, d2)
        plsc.subcore_barrier()
        def body(j, _):
            pltpu.sync_copy(edge_hbm.at[pl.ds(tbase + j * CH, CH)], rows)
            pltpu.sync_copy(rows, acc.at[d2.at[j]], add=True)
            return 0
        lax.fori_loop(0, NCH, body, 0)
        plsc.subcore_barrier()
        pltpu.sync_copy(acc.at[pl.ds(s * ZPT, ZPT)],
                        out_hbm.at[pl.ds(c * NPAD + s * ZPT, ZPT)])

    return k(edge_rep, dst), NPAD


def _sc_agg_c(edge_rep, c2e_edge, c2e_cycle, n_cycles):
    E, D = edge_rep.shape
    M = c2e_edge.shape[0]
    MPT = M // (_NC * _NS)
    CH = 128
    NCH = MPT // CH
    ZPT = n_cycles // _NS

    @functools.partial(
        pl.kernel,
        out_type=jax.ShapeDtypeStruct((_NC * n_cycles, D), jnp.float32),
        mesh=_mesh(),
        scratch_types=[
            pltpu.VMEM_SHARED((n_cycles, D), jnp.float32),
            pltpu.VMEM((MPT,), jnp.int32),
            pltpu.VMEM((MPT,), jnp.int32),
            pltpu.VMEM((NCH, CH), jnp.int32),
            pltpu.VMEM((CH, D), jnp.float32),
            pltpu.VMEM((32, D), jnp.float32),
        ],
    )
    def k(edge_hbm, ce_hbm, cc_hbm, out_hbm, acc, gbuf, cbuf, s2, rows, zbuf):
        c = lax.axis_index("c")
        s = lax.axis_index("s")
        _zero_zbuf(zbuf)
        _zero_acc_range(acc, zbuf, s * ZPT, ZPT)
        mbase = (c * _NS + s) * MPT
        pltpu.sync_copy(ce_hbm.at[pl.ds(mbase, MPT)], gbuf)
        pltpu.sync_copy(cc_hbm.at[pl.ds(mbase, MPT)], cbuf)
        _repack(cbuf, s2)
        plsc.subcore_barrier()
        def body(j, _):
            pltpu.sync_copy(edge_hbm.at[gbuf.at[pl.ds(j * CH, CH)]], rows)
            pltpu.sync_copy(rows, acc.at[s2.at[j]], add=True)
            return 0
        lax.fori_loop(0, NCH, body, 0)
        plsc.subcore_barrier()
        pltpu.sync_copy(acc.at[pl.ds(s * ZPT, ZPT)],
                        out_hbm.at[pl.ds(c * n_cycles + s * ZPT, ZPT)])

    return k(edge_rep, c2e_edge, c2e_cycle)


def _sc_a1(pa, pb, src, dst, n_edges):
    D = pa.shape[1]
    EPT = n_edges // (_NC * _NS)
    CH = 80
    NCH = EPT // CH

    @functools.partial(
        pl.kernel,
        out_type=jax.ShapeDtypeStruct((n_edges, D), jnp.float32),
        mesh=_mesh(),
        scratch_types=[
            pltpu.VMEM((EPT,), jnp.int32),
            pltpu.VMEM((EPT,), jnp.int32),
            pltpu.VMEM((CH, D), jnp.float32),
        ],
    )
    def k(pa_hbm, pb_hbm, src_hbm, dst_hbm, out_hbm, sbuf, dbuf, rows):
        c = lax.axis_index("c")
        s = lax.axis_index("s")
        ebase = (c * _NS + s) * EPT
        pltpu.sync_copy(src_hbm.at[pl.ds(ebase, EPT)], sbuf)
        pltpu.sync_copy(dst_hbm.at[pl.ds(ebase, EPT)], dbuf)
        def body(j, _):
            pltpu.sync_copy(pa_hbm.at[sbuf.at[pl.ds(j * CH, CH)]], rows)
            pltpu.sync_copy(pb_hbm.at[dbuf.at[pl.ds(j * CH, CH)]], rows,
                            add=True)
            pltpu.sync_copy(rows, out_hbm.at[pl.ds(ebase + j * CH, CH)])
            return 0
        lax.fori_loop(0, NCH, body, 0)

    return k(pa, pb, src, dst)


def _sc_scatter_s(cw, c2e_edge, c2e_cycle, n_edges):
    D = cw.shape[1]
    M = c2e_edge.shape[0]
    RS = 12032
    ACC_ROWS = RS + 128
    NW = -(-n_edges // RS)
    NPASS = -(-NW // _NC)
    MPT = M // _NS
    CH = 128
    NCH = MPT // CH
    ZPT = ACC_ROWS // _NS
    DPT = RS // _NS

    @functools.partial(
        pl.kernel,
        out_type=jax.ShapeDtypeStruct((n_edges, D), jnp.float32),
        mesh=_mesh(),
        scratch_types=[
            pltpu.VMEM_SHARED((ACC_ROWS, D), jnp.float32),
            pltpu.VMEM((MPT,), jnp.int32),
            pltpu.VMEM((MPT,), jnp.int32),
            pltpu.VMEM((NCH, CH), jnp.int32),
            pltpu.VMEM((CH, D), jnp.float32),
            pltpu.VMEM((32, D), jnp.float32),
        ],
    )
    def k(cw_hbm, ce_hbm, cc_hbm, out_hbm, acc, ebuf, cbuf, li, rows, zbuf):
        c = lax.axis_index("c")
        s = lax.axis_index("s")
        _zero_zbuf(zbuf)
        mbase = s * MPT
        pltpu.sync_copy(ce_hbm.at[pl.ds(mbase, MPT)], ebuf)
        pltpu.sync_copy(cc_hbm.at[pl.ds(mbase, MPT)], cbuf)
        for p in range(NPASS):
            base = (_NC * p + c) * RS
            _zero_acc_range(acc, zbuf, s * ZPT, ZPT)
            def cli(jr, _):
                for u in range(CH // _L):
                    e = ebuf[pl.ds(jr * CH + u * _L, _L)]
                    inr = (e >= base) & (e < base + RS)
                    li[jr, pl.ds(u * _L, _L)] = jnp.where(inr, e - base,
                                                          RS + s * 8)
                return 0
            lax.fori_loop(0, NCH, cli, 0)
            plsc.subcore_barrier()
            def sct(j, _):
                pltpu.sync_copy(cw_hbm.at[cbuf.at[pl.ds(j * CH, CH)]], rows)
                pltpu.sync_copy(rows, acc.at[li.at[j]], add=True)
                return 0
            lax.fori_loop(0, NCH, sct, 0)
            plsc.subcore_barrier()
            for cc in range(_NC):
                w = _NC * p + cc
                if w >= NW:
                    continue
                wbase = w * RS
                wrows = min(RS, n_edges - wbase)
                dpt = wrows // _NS
                @pl.when(c == cc)
                def _():
                    pltpu.sync_copy(
                        acc.at[pl.ds(s * dpt, dpt)],
                        out_hbm.at[pl.ds(wbase + s * dpt, dpt)])
            plsc.subcore_barrier()

    return k(cw, c2e_edge, c2e_cycle)


def _mm_body(x_ref, w_ref, out_ref):
    out_ref[...] = jnp.dot(x_ref[...], w_ref[...],
                           preferred_element_type=jnp.float32)


def _mm(x, w, block):
    n, d = x.shape
    d2 = w.shape[1]
    return pl.pallas_call(
        _mm_body,
        grid=(n // block,),
        in_specs=[pl.BlockSpec((block, d), lambda i: (i, 0)),
                  pl.BlockSpec((d, d2), lambda i: (0, 0))],
        out_specs=pl.BlockSpec((block, d2), lambda i: (i, 0)),
        out_shape=jax.ShapeDtypeStruct((n, d2), jnp.float32),
    )(x, w)


def _node_body(x_ref, a0_ref, a1_ref, w0_ref, w1_ref, w2_ref, out_ref):
    h = x_ref[...] + jnp.dot(a0_ref[...] + a1_ref[...], w0_ref[...],
                             preferred_element_type=jnp.float32)
    h = jnp.maximum(jnp.dot(h, w1_ref[...],
                            preferred_element_type=jnp.float32), 0.0)
    out_ref[...] = jnp.dot(h, w2_ref[...], preferred_element_type=jnp.float32)


def _node_mlp(x, a0, a1, w0, w1, w2, block):
    n, d = x.shape
    bs = pl.BlockSpec((block, d), lambda i: (i, 0))
    ws = pl.BlockSpec((d, d), lambda i: (0, 0))
    return pl.pallas_call(
        _node_body,
        grid=(n // block,),
        in_specs=[bs, bs, bs, ws, ws, ws],
        out_specs=bs,
        out_shape=jax.ShapeDtypeStruct((n, d), jnp.float32),
    )(x, a0, a1, w0, w1, w2)


def _gin_body(x_ref, a0_ref, a1_ref, w1_ref, w2_ref, out_ref):
    h = x_ref[...] + a0_ref[...] + a1_ref[...]
    h = jnp.maximum(jnp.dot(h, w1_ref[...],
                            preferred_element_type=jnp.float32), 0.0)
    out_ref[...] = jnp.dot(h, w2_ref[...], preferred_element_type=jnp.float32)


def _gin(x, a0, a1, w1, w2, block):
    n, d = x.shape
    bs = pl.BlockSpec((block, d), lambda i: (i, 0))
    ws = pl.BlockSpec((d, d), lambda i: (0, 0))
    return pl.pallas_call(
        _gin_body,
        grid=(n // block,),
        in_specs=[bs, bs, bs, ws, ws],
        out_specs=bs,
        out_shape=jax.ShapeDtypeStruct((n, d), jnp.float32),
    )(x, a0, a1, w1, w2)


def _edge_body(edge_ref, a1_ref, s_ref, wec1_ref, wec2_ref, wm1a_ref,
               wm1b_ref, wm2_ref, out_ref):
    e1 = jnp.maximum(a1_ref[...], 0.0)
    t = jnp.dot(edge_ref[...], wec1_ref[...],
                preferred_element_type=jnp.float32) + s_ref[...]
    e2 = jnp.dot(jnp.maximum(t, 0.0), wec2_ref[...],
                 preferred_element_type=jnp.float32)
    m = jnp.dot(e1, wm1a_ref[...], preferred_element_type=jnp.float32)
    m += jnp.dot(e2, wm1b_ref[...], preferred_element_type=jnp.float32)
    out_ref[...] = jnp.dot(jnp.maximum(m, 0.0), wm2_ref[...],
                           preferred_element_type=jnp.float32)


def _edge_mlp(edge_rep, a1, s, wec1s, wec2, wm1a, wm1b, wm2, block):
    e, d = edge_rep.shape
    bs = pl.BlockSpec((block, d), lambda i: (i, 0))
    ws = pl.BlockSpec((d, d), lambda i: (0, 0))
    return pl.pallas_call(
        _edge_body,
        grid=(e // block,),
        in_specs=[bs, bs, bs, ws, ws, ws, ws, ws],
        out_specs=bs,
        out_shape=jax.ShapeDtypeStruct((e, d), jnp.float32),
    )(edge_rep, a1, s, wec1s, wec2, wm1a, wm1b, wm2)


def kernel(node_rep, edge_rep, cycle_rep, edge_index, c2e_edge, c2e_cycle,
           W_e2n, W_n1, W_n2, W_ne1, W_ec1, W_ec2, W_ce1, W_ce2, W_m1, W_m2,
           eps_e, eps_c):
    n_nodes, d = node_rep.shape
    n_edges = edge_rep.shape[0]
    n_cycles = cycle_rep.shape[0]
    src = edge_index[0]
    dst = edge_index[1]

    W_a, W_b = W_ne1[:d], W_ne1[d:]
    pa = _mm(node_rep, W_a, block=1000)
    pb = _mm(node_rep, W_b, block=1000)
    cw = _mm(cycle_rep, W_ec1, block=1024)

    aggn2, npad = _sc_agg_n(edge_rep, dst, n_nodes)
    aggc2 = _sc_agg_c(edge_rep, c2e_edge, c2e_cycle, n_cycles)
    a1 = _sc_a1(pa, pb, src, dst, n_edges)
    s = _sc_scatter_s(cw, c2e_edge, c2e_cycle, n_edges)

    node_out = _node_mlp(node_rep, aggn2[:n_nodes], aggn2[npad:npad + n_nodes],
                         W_e2n, W_n1, W_n2, block=1000)
    cin = (1.0 + eps_c) * cycle_rep
    cycle_out = _gin(cin, aggc2[:n_cycles], aggc2[n_cycles:],
                     W_ce1, W_ce2, block=1024)
    wec1s = (1.0 + eps_e) * W_ec1
    W_m1a, W_m1b = W_m1[:d], W_m1[d:]
    edge_out = _edge_mlp(edge_rep, a1, s, wec1s, W_ec2, W_m1a, W_m1b, W_m2,
                         block=2000)
    return (node_out, edge_out, cycle_out)


# speedup vs baseline: 1.1202x; 1.0631x over previous
import functools

import jax
import jax.numpy as jnp
from jax import lax
from jax.experimental import pallas as pl
from jax.experimental.pallas import tpu as pltpu
from jax.experimental.pallas import tpu_sc as plsc

_NC = 2
_NS = 16
_L = 16


def _mesh():
    return plsc.VectorSubcoreMesh(core_axis_name="c", subcore_axis_name="s")


def _zero_zbuf(zbuf):
    def zrow(i, _):
        for u in range(zbuf.shape[1] // _L):
            zbuf[i, pl.ds(u * _L, _L)] = jnp.zeros((_L,), jnp.float32)
        return 0
    lax.fori_loop(0, zbuf.shape[0], zrow, 0)


def _zero_acc_range(acc, zbuf, start, n):
    z = zbuf.shape[0]
    n_full, rem = n // z, n % z
    def zacc(i, _):
        pltpu.sync_copy(zbuf, acc.at[pl.ds(start + i * z, z)])
        return 0
    lax.fori_loop(0, n_full, zacc, 0)
    if rem:
        pltpu.sync_copy(zbuf.at[pl.ds(0, rem)],
                        acc.at[pl.ds(start + n_full * z, rem)])


def _repack(src1d, dst2d):
    nch, ch = dst2d.shape
    def rp(j, _):
        for u in range(ch // _L):
            dst2d[j, pl.ds(u * _L, _L)] = src1d[pl.ds(j * ch + u * _L, _L)]
        return 0
    lax.fori_loop(0, nch, rp, 0)


def _sc_agg_n(edge_rep, dst, n_nodes):
    E, D = edge_rep.shape
    EPT = E // (_NC * _NS)
    CH = 80
    NCH = EPT // CH
    NPAD = -(-n_nodes // 128) * 128
    ZPT = NPAD // _NS

    @functools.partial(
        pl.kernel,
        out_type=jax.ShapeDtypeStruct((_NC * NPAD, D), jnp.float32),
        mesh=_mesh(),
        scratch_types=[
            pltpu.VMEM_SHARED((NPAD, D), jnp.float32),
            pltpu.VMEM((EPT,), jnp.int32),
            pltpu.VMEM((NCH, CH), jnp.int32),
            pltpu.VMEM((CH, D), jnp.float32),
            pltpu.VMEM((CH, D), jnp.float32),
            pltpu.SemaphoreType.DMA,
            pltpu.SemaphoreType.DMA,
        ],
    )
    def k(edge_hbm, dst_hbm, out_hbm, acc, dbuf, d2, rows_a, rows_b,
          sem_a, sem_b):
        c = lax.axis_index("c")
        s = lax.axis_index("s")
        _zero_zbuf(rows_a)
        _zero_acc_range(acc, rows_a, s * ZPT, ZPT)
        tbase = (c * _NS + s) * EPT
        pltpu.sync_copy(dst_hbm.at[pl.ds(tbase, EPT)], dbuf)
        _repack(dbuf, d2)
        plsc.subcore_barrier()
        def body(j2, _):
            j = 2 * j2
            ga = pltpu.async_copy(edge_hbm.at[pl.ds(tbase + j * CH, CH)],
                                  rows_a, sem_a)
            gb = pltpu.async_copy(edge_hbm.at[pl.ds(tbase + (j + 1) * CH, CH)],
                                  rows_b, sem_b)
            ga.wait()
            pltpu.sync_copy(rows_a, acc.at[d2.at[j]], add=True)
            gb.wait()
            pltpu.sync_copy(rows_b, acc.at[d2.at[j + 1]], add=True)
            return 0
        lax.fori_loop(0, NCH // 2, body, 0)
        if NCH % 2:
            j = NCH - 1
            pltpu.sync_copy(edge_hbm.at[pl.ds(tbase + j * CH, CH)], rows_a)
            pltpu.sync_copy(rows_a, acc.at[d2.at[j]], add=True)
        plsc.subcore_barrier()
        pltpu.sync_copy(acc.at[pl.ds(s * ZPT, ZPT)],
                        out_hbm.at[pl.ds(c * NPAD + s * ZPT, ZPT)])

    return k(edge_rep, dst), NPAD


def _sc_agg_c(edge_rep, c2e_edge, c2e_cycle, n_cycles):
    E, D = edge_rep.shape
    M = c2e_edge.shape[0]
    MPT = M // (_NC * _NS)
    CH = 128
    NCH = MPT // CH
    ZPT = n_cycles // _NS

    @functools.partial(
        pl.kernel,
        out_type=jax.ShapeDtypeStruct((_NC * n_cycles, D), jnp.float32),
        mesh=_mesh(),
        scratch_types=[
            pltpu.VMEM_SHARED((n_cycles, D), jnp.float32),
            pltpu.VMEM((MPT,), jnp.int32),
            pltpu.VMEM((MPT,), jnp.int32),
            pltpu.VMEM((NCH, CH), jnp.int32),
            pltpu.VMEM((CH, D), jnp.float32),
            pltpu.VMEM((32, D), jnp.float32),
        ],
    )
    def k(edge_hbm, ce_hbm, cc_hbm, out_hbm, acc, gbuf, cbuf, s2, rows, zbuf):
        c = lax.axis_index("c")
        s = lax.axis_index("s")
        _zero_zbuf(zbuf)
        _zero_acc_range(acc, zbuf, s * ZPT, ZPT)
        mbase = (c * _NS + s) * MPT
        pltpu.sync_copy(ce_hbm.at[pl.ds(mbase, MPT)], gbuf)
        pltpu.sync_copy(cc_hbm.at[pl.ds(mbase, MPT)], cbuf)
        _repack(cbuf, s2)
        plsc.subcore_barrier()
        def body(j, _):
            pltpu.sync_copy(edge_hbm.at[gbuf.at[pl.ds(j * CH, CH)]], rows)
            pltpu.sync_copy(rows, acc.at[s2.at[j]], add=True)
            return 0
        lax.fori_loop(0, NCH, body, 0)
        plsc.subcore_barrier()
        pltpu.sync_copy(acc.at[pl.ds(s * ZPT, ZPT)],
                        out_hbm.at[pl.ds(c * n_cycles + s * ZPT, ZPT)])

    return k(edge_rep, c2e_edge, c2e_cycle)


def _sc_a1(pa, pb, src, dst, n_edges):
    D = pa.shape[1]
    EPT = n_edges // (_NC * _NS)
    CH = 80
    NCH = EPT // CH

    @functools.partial(
        pl.kernel,
        out_type=jax.ShapeDtypeStruct((n_edges, D), jnp.float32),
        mesh=_mesh(),
        scratch_types=[
            pltpu.VMEM((EPT,), jnp.int32),
            pltpu.VMEM((EPT,), jnp.int32),
            pltpu.VMEM((CH, D), jnp.float32),
            pltpu.VMEM((CH, D), jnp.float32),
            pltpu.SemaphoreType.DMA,
            pltpu.SemaphoreType.DMA,
        ],
    )
    def k(pa_hbm, pb_hbm, src_hbm, dst_hbm, out_hbm, sbuf, dbuf,
          rows_a, rows_b, sem_a, sem_b):
        c = lax.axis_index("c")
        s = lax.axis_index("s")
        ebase = (c * _NS + s) * EPT
        pltpu.sync_copy(src_hbm.at[pl.ds(ebase, EPT)], sbuf)
        pltpu.sync_copy(dst_hbm.at[pl.ds(ebase, EPT)], dbuf)
        def chain(j, rows, sem):
            g1 = pltpu.async_copy(pa_hbm.at[sbuf.at[pl.ds(j * CH, CH)]],
                                  rows, sem)
            return g1
        def body(j2, _):
            j = 2 * j2
            g1 = chain(j, rows_a, sem_a)
            g2 = chain(j + 1, rows_b, sem_b)
            g1.wait()
            a1 = pltpu.async_copy(pb_hbm.at[dbuf.at[pl.ds(j * CH, CH)]],
                                  rows_a, sem_a, add=True)
            g2.wait()
            a2 = pltpu.async_copy(pb_hbm.at[dbuf.at[pl.ds((j + 1) * CH, CH)]],
                                  rows_b, sem_b, add=True)
            a1.wait()
            o1 = pltpu.async_copy(rows_a, out_hbm.at[pl.ds(ebase + j * CH, CH)],
                                  sem_a)
            a2.wait()
            o2 = pltpu.async_copy(rows_b,
                                  out_hbm.at[pl.ds(ebase + (j + 1) * CH, CH)],
                                  sem_b)
            o1.wait()
            o2.wait()
            return 0
        lax.fori_loop(0, NCH // 2, body, 0)
        if NCH % 2:
            j = NCH - 1
            pltpu.sync_copy(pa_hbm.at[sbuf.at[pl.ds(j * CH, CH)]], rows_a)
            pltpu.sync_copy(pb_hbm.at[dbuf.at[pl.ds(j * CH, CH)]], rows_a,
                            add=True)
            pltpu.sync_copy(rows_a, out_hbm.at[pl.ds(ebase + j * CH, CH)])

    return k(pa, pb, src, dst)


def _sc_scatter_s(cw, c2e_edge, c2e_cycle, n_edges):
    D = cw.shape[1]
    M = c2e_edge.shape[0]
    RS = 10496
    ACC_ROWS = RS + 128
    NW = -(-n_edges // RS)
    NPASS = -(-NW // _NC)
    MPT = M // _NS
    CH = 128
    NCH = MPT // CH
    ZPT = ACC_ROWS // _NS

    @functools.partial(
        pl.kernel,
        out_type=jax.ShapeDtypeStruct((n_edges, D), jnp.float32),
        mesh=_mesh(),
        scratch_types=[
            pltpu.VMEM_SHARED((ACC_ROWS, D), jnp.float32),
            pltpu.VMEM((MPT,), jnp.int32),
            pltpu.VMEM((MPT,), jnp.int32),
            pltpu.VMEM((NCH, CH), jnp.int32),
            pltpu.VMEM((CH, D), jnp.float32),
            pltpu.VMEM((CH, D), jnp.float32),
            pltpu.SemaphoreType.DMA,
            pltpu.SemaphoreType.DMA,
        ],
    )
    def k(cw_hbm, ce_hbm, cc_hbm, out_hbm, acc, ebuf, cbuf, li,
          rows_a, rows_b, sem_a, sem_b):
        c = lax.axis_index("c")
        s = lax.axis_index("s")
        mbase = s * MPT
        pltpu.sync_copy(ce_hbm.at[pl.ds(mbase, MPT)], ebuf)
        pltpu.sync_copy(cc_hbm.at[pl.ds(mbase, MPT)], cbuf)
        for p in range(NPASS):
            base = (_NC * p + c) * RS
            _zero_zbuf(rows_a)
            _zero_acc_range(acc, rows_a, s * ZPT, ZPT)
            def cli(jr, _):
                for u in range(CH // _L):
                    e = ebuf[pl.ds(jr * CH + u * _L, _L)]
                    inr = (e >= base) & (e < base + RS)
                    li[jr, pl.ds(u * _L, _L)] = jnp.where(inr, e - base,
                                                          RS + s * 8)
                return 0
            lax.fori_loop(0, NCH, cli, 0)
            plsc.subcore_barrier()
            def sct(j2, _):
                j = 2 * j2
                ga = pltpu.async_copy(cw_hbm.at[cbuf.at[pl.ds(j * CH, CH)]],
                                      rows_a, sem_a)
                gb = pltpu.async_copy(
                    cw_hbm.at[cbuf.at[pl.ds((j + 1) * CH, CH)]], rows_b, sem_b)
                ga.wait()
                pltpu.sync_copy(rows_a, acc.at[li.at[j]], add=True)
                gb.wait()
                pltpu.sync_copy(rows_b, acc.at[li.at[j + 1]], add=True)
                return 0
            lax.fori_loop(0, NCH // 2, sct, 0)
            plsc.subcore_barrier()
            for cc in range(_NC):
                w = _NC * p + cc
                if w >= NW:
                    continue
                wbase = w * RS
                wrows = min(RS, n_edges - wbase)
                dpt = wrows // _NS
                @pl.when(c == cc)
                def _():
                    pltpu.sync_copy(
                        acc.at[pl.ds(s * dpt, dpt)],
                        out_hbm.at[pl.ds(wbase + s * dpt, dpt)])
            plsc.subcore_barrier()

    return k(cw, c2e_edge, c2e_cycle)


def _mm_body(x_ref, w_ref, out_ref):
    out_ref[...] = jnp.dot(x_ref[...], w_ref[...],
                           preferred_element_type=jnp.float32)


def _mm(x, w, block):
    n, d = x.shape
    d2 = w.shape[1]
    return pl.pallas_call(
        _mm_body,
        grid=(n // block,),
        in_specs=[pl.BlockSpec((block, d), lambda i: (i, 0)),
                  pl.BlockSpec((d, d2), lambda i: (0, 0))],
        out_specs=pl.BlockSpec((block, d2), lambda i: (i, 0)),
        out_shape=jax.ShapeDtypeStruct((n, d2), jnp.float32),
    )(x, w)


def _node_body(x_ref, a0_ref, a1_ref, w0_ref, w1_ref, w2_ref, out_ref):
    h = x_ref[...] + jnp.dot(a0_ref[...] + a1_ref[...], w0_ref[...],
                             preferred_element_type=jnp.float32)
    h = jnp.maximum(jnp.dot(h, w1_ref[...],
                            preferred_element_type=jnp.float32), 0.0)
    out_ref[...] = jnp.dot(h, w2_ref[...], preferred_element_type=jnp.float32)


def _node_mlp(x, a0, a1, w0, w1, w2, block):
    n, d = x.shape
    bs = pl.BlockSpec((block, d), lambda i: (i, 0))
    ws = pl.BlockSpec((d, d), lambda i: (0, 0))
    return pl.pallas_call(
        _node_body,
        grid=(n // block,),
        in_specs=[bs, bs, bs, ws, ws, ws],
        out_specs=bs,
        out_shape=jax.ShapeDtypeStruct((n, d), jnp.float32),
    )(x, a0, a1, w0, w1, w2)


def _gin_body(x_ref, a0_ref, a1_ref, w1_ref, w2_ref, out_ref):
    h = x_ref[...] + a0_ref[...] + a1_ref[...]
    h = jnp.maximum(jnp.dot(h, w1_ref[...],
                            preferred_element_type=jnp.float32), 0.0)
    out_ref[...] = jnp.dot(h, w2_ref[...], preferred_element_type=jnp.float32)


def _gin(x, a0, a1, w1, w2, block):
    n, d = x.shape
    bs = pl.BlockSpec((block, d), lambda i: (i, 0))
    ws = pl.BlockSpec((d, d), lambda i: (0, 0))
    return pl.pallas_call(
        _gin_body,
        grid=(n // block,),
        in_specs=[bs, bs, bs, ws, ws],
        out_specs=bs,
        out_shape=jax.ShapeDtypeStruct((n, d), jnp.float32),
    )(x, a0, a1, w1, w2)


def _edge_body(edge_ref, a1_ref, s_ref, wec1_ref, wec2_ref, wm1a_ref,
               wm1b_ref, wm2_ref, out_ref):
    e1 = jnp.maximum(a1_ref[...], 0.0)
    t = jnp.dot(edge_ref[...], wec1_ref[...],
                preferred_element_type=jnp.float32) + s_ref[...]
    e2 = jnp.dot(jnp.maximum(t, 0.0), wec2_ref[...],
                 preferred_element_type=jnp.float32)
    m = jnp.dot(e1, wm1a_ref[...], preferred_element_type=jnp.float32)
    m += jnp.dot(e2, wm1b_ref[...], preferred_element_type=jnp.float32)
    out_ref[...] = jnp.dot(jnp.maximum(m, 0.0), wm2_ref[...],
                           preferred_element_type=jnp.float32)


def _edge_mlp(edge_rep, a1, s, wec1s, wec2, wm1a, wm1b, wm2, block):
    e, d = edge_rep.shape
    bs = pl.BlockSpec((block, d), lambda i: (i, 0))
    ws = pl.BlockSpec((d, d), lambda i: (0, 0))
    return pl.pallas_call(
        _edge_body,
        grid=(e // block,),
        in_specs=[bs, bs, bs, ws, ws, ws, ws, ws],
        out_specs=bs,
        out_shape=jax.ShapeDtypeStruct((e, d), jnp.float32),
    )(edge_rep, a1, s, wec1s, wec2, wm1a, wm1b, wm2)


def kernel(node_rep, edge_rep, cycle_rep, edge_index, c2e_edge, c2e_cycle,
           W_e2n, W_n1, W_n2, W_ne1, W_ec1, W_ec2, W_ce1, W_ce2, W_m1, W_m2,
           eps_e, eps_c):
    n_nodes, d = node_rep.shape
    n_edges = edge_rep.shape[0]
    n_cycles = cycle_rep.shape[0]
    src = edge_index[0]
    dst = edge_index[1]

    W_a, W_b = W_ne1[:d], W_ne1[d:]
    pa = _mm(node_rep, W_a, block=1000)
    pb = _mm(node_rep, W_b, block=1000)
    cw = _mm(cycle_rep, W_ec1, block=1024)

    aggn2, npad = _sc_agg_n(edge_rep, dst, n_nodes)
    aggc2 = _sc_agg_c(edge_rep, c2e_edge, c2e_cycle, n_cycles)
    a1 = _sc_a1(pa, pb, src, dst, n_edges)
    s = _sc_scatter_s(cw, c2e_edge, c2e_cycle, n_edges)

    node_out = _node_mlp(node_rep, aggn2[:n_nodes], aggn2[npad:npad + n_nodes],
                         W_e2n, W_n1, W_n2, block=1000)
    cin = (1.0 + eps_c) * cycle_rep
    cycle_out = _gin(cin, aggc2[:n_cycles], aggc2[n_cycles:],
                     W_ce1, W_ce2, block=1024)
    wec1s = (1.0 + eps_e) * W_ec1
    W_m1a, W_m1b = W_m1[:d], W_m1[d:]
    edge_out = _edge_mlp(edge_rep, a1, s, wec1s, W_ec2, W_m1a, W_m1b, W_m2,
                         block=2000)
    return (node_out, edge_out, cycle_out)


# speedup vs baseline: 1.1700x; 1.0445x over previous
import functools

import jax
import jax.numpy as jnp
from jax import lax
from jax.experimental import pallas as pl
from jax.experimental.pallas import tpu as pltpu
from jax.experimental.pallas import tpu_sc as plsc

_NC = 2
_NS = 16
_L = 16


def _mesh():
    return plsc.VectorSubcoreMesh(core_axis_name="c", subcore_axis_name="s")


def _zero_zbuf(zbuf):
    def zrow(i, _):
        for u in range(zbuf.shape[1] // _L):
            zbuf[i, pl.ds(u * _L, _L)] = jnp.zeros((_L,), jnp.float32)
        return 0
    lax.fori_loop(0, zbuf.shape[0], zrow, 0)


def _zero_acc_range(acc, zbuf, start, n):
    z = zbuf.shape[0]
    n_full, rem = n // z, n % z
    def zacc(i, _):
        pltpu.sync_copy(zbuf, acc.at[pl.ds(start + i * z, z)])
        return 0
    lax.fori_loop(0, n_full, zacc, 0)
    if rem:
        pltpu.sync_copy(zbuf.at[pl.ds(0, rem)],
                        acc.at[pl.ds(start + n_full * z, rem)])


def _repack(src1d, dst2d):
    nch, ch = dst2d.shape
    def rp(j, _):
        for u in range(ch // _L):
            dst2d[j, pl.ds(u * _L, _L)] = src1d[pl.ds(j * ch + u * _L, _L)]
        return 0
    lax.fori_loop(0, nch, rp, 0)


def _sc_agg_n(edge_rep, dst, n_nodes):
    E, D = edge_rep.shape
    EPT = E // (_NC * _NS)
    CH = 80
    NCH = EPT // CH
    NPAD = -(-n_nodes // 128) * 128
    ZPT = NPAD // _NS

    @functools.partial(
        pl.kernel,
        out_type=jax.ShapeDtypeStruct((_NC * NPAD, D), jnp.float32),
        mesh=_mesh(),
        scratch_types=[
            pltpu.VMEM_SHARED((NPAD, D), jnp.float32),
            pltpu.VMEM((EPT,), jnp.int32),
            pltpu.VMEM((NCH, CH), jnp.int32),
            pltpu.VMEM((CH, D), jnp.float32),
            pltpu.VMEM((CH, D), jnp.float32),
            pltpu.SemaphoreType.DMA,
            pltpu.SemaphoreType.DMA,
        ],
    )
    def k(edge_hbm, dst_hbm, out_hbm, acc, dbuf, d2, rows_a, rows_b,
          sem_a, sem_b):
        c = lax.axis_index("c")
        s = lax.axis_index("s")
        _zero_zbuf(rows_a)
        _zero_acc_range(acc, rows_a, s * ZPT, ZPT)
        tbase = (c * _NS + s) * EPT
        pltpu.sync_copy(dst_hbm.at[pl.ds(tbase, EPT)], dbuf)
        _repack(dbuf, d2)
        plsc.subcore_barrier()
        def body(j2, _):
            j = 2 * j2
            ga = pltpu.async_copy(edge_hbm.at[pl.ds(tbase + j * CH, CH)],
                                  rows_a, sem_a)
            gb = pltpu.async_copy(edge_hbm.at[pl.ds(tbase + (j + 1) * CH, CH)],
                                  rows_b, sem_b)
            ga.wait()
            pltpu.sync_copy(rows_a, acc.at[d2.at[j]], add=True)
            gb.wait()
            pltpu.sync_copy(rows_b, acc.at[d2.at[j + 1]], add=True)
            return 0
        lax.fori_loop(0, NCH // 2, body, 0)
        if NCH % 2:
            j = NCH - 1
            pltpu.sync_copy(edge_hbm.at[pl.ds(tbase + j * CH, CH)], rows_a)
            pltpu.sync_copy(rows_a, acc.at[d2.at[j]], add=True)
        plsc.subcore_barrier()
        pltpu.sync_copy(acc.at[pl.ds(s * ZPT, ZPT)],
                        out_hbm.at[pl.ds(c * NPAD + s * ZPT, ZPT)])

    return k(edge_rep, dst), NPAD


def _sc_agg_c(edge_rep, c2e_edge, c2e_cycle, n_cycles):
    E, D = edge_rep.shape
    M = c2e_edge.shape[0]
    MPT = M // (_NC * _NS)
    CH = 128
    NCH = MPT // CH
    ZPT = n_cycles // _NS

    @functools.partial(
        pl.kernel,
        out_type=jax.ShapeDtypeStruct((_NC * n_cycles, D), jnp.float32),
        mesh=_mesh(),
        scratch_types=[
            pltpu.VMEM_SHARED((n_cycles, D), jnp.float32),
            pltpu.VMEM((MPT,), jnp.int32),
            pltpu.VMEM((MPT,), jnp.int32),
            pltpu.VMEM((NCH, CH), jnp.int32),
            pltpu.VMEM((CH, D), jnp.float32),
            pltpu.VMEM((32, D), jnp.float32),
        ],
    )
    def k(edge_hbm, ce_hbm, cc_hbm, out_hbm, acc, gbuf, cbuf, s2, rows, zbuf):
        c = lax.axis_index("c")
        s = lax.axis_index("s")
        _zero_zbuf(zbuf)
        _zero_acc_range(acc, zbuf, s * ZPT, ZPT)
        mbase = (c * _NS + s) * MPT
        pltpu.sync_copy(ce_hbm.at[pl.ds(mbase, MPT)], gbuf)
        pltpu.sync_copy(cc_hbm.at[pl.ds(mbase, MPT)], cbuf)
        _repack(cbuf, s2)
        plsc.subcore_barrier()
        def body(j, _):
            pltpu.sync_copy(edge_hbm.at[gbuf.at[pl.ds(j * CH, CH)]], rows)
            pltpu.sync_copy(rows, acc.at[s2.at[j]], add=True)
            return 0
        lax.fori_loop(0, NCH, body, 0)
        plsc.subcore_barrier()
        pltpu.sync_copy(acc.at[pl.ds(s * ZPT, ZPT)],
                        out_hbm.at[pl.ds(c * n_cycles + s * ZPT, ZPT)])

    return k(edge_rep, c2e_edge, c2e_cycle)


def _sc_a1(pa, pb, src, dst, n_edges):
    D = pa.shape[1]
    EPT = n_edges // (_NC * _NS)
    CH = 80
    NCH = EPT // CH

    @functools.partial(
        pl.kernel,
        out_type=jax.ShapeDtypeStruct((n_edges, D), jnp.float32),
        mesh=_mesh(),
        scratch_types=[
            pltpu.VMEM((EPT,), jnp.int32),
            pltpu.VMEM((EPT,), jnp.int32),
            pltpu.VMEM((CH, D), jnp.float32),
            pltpu.VMEM((CH, D), jnp.float32),
            pltpu.SemaphoreType.DMA,
            pltpu.SemaphoreType.DMA,
        ],
    )
    def k(pa_hbm, pb_hbm, src_hbm, dst_hbm, out_hbm, sbuf, dbuf,
          rows_a, rows_b, sem_a, sem_b):
        c = lax.axis_index("c")
        s = lax.axis_index("s")
        ebase = (c * _NS + s) * EPT
        pltpu.sync_copy(src_hbm.at[pl.ds(ebase, EPT)], sbuf)
        pltpu.sync_copy(dst_hbm.at[pl.ds(ebase, EPT)], dbuf)
        def chain(j, rows, sem):
            g1 = pltpu.async_copy(pa_hbm.at[sbuf.at[pl.ds(j * CH, CH)]],
                                  rows, sem)
            return g1
        def body(j2, _):
            j = 2 * j2
            g1 = chain(j, rows_a, sem_a)
            g2 = chain(j + 1, rows_b, sem_b)
            g1.wait()
            a1 = pltpu.async_copy(pb_hbm.at[dbuf.at[pl.ds(j * CH, CH)]],
                                  rows_a, sem_a, add=True)
            g2.wait()
            a2 = pltpu.async_copy(pb_hbm.at[dbuf.at[pl.ds((j + 1) * CH, CH)]],
                                  rows_b, sem_b, add=True)
            a1.wait()
            o1 = pltpu.async_copy(rows_a, out_hbm.at[pl.ds(ebase + j * CH, CH)],
                                  sem_a)
            a2.wait()
            o2 = pltpu.async_copy(rows_b,
                                  out_hbm.at[pl.ds(ebase + (j + 1) * CH, CH)],
                                  sem_b)
            o1.wait()
            o2.wait()
            return 0
        lax.fori_loop(0, NCH // 2, body, 0)
        if NCH % 2:
            j = NCH - 1
            pltpu.sync_copy(pa_hbm.at[sbuf.at[pl.ds(j * CH, CH)]], rows_a)
            pltpu.sync_copy(pb_hbm.at[dbuf.at[pl.ds(j * CH, CH)]], rows_a,
                            add=True)
            pltpu.sync_copy(rows_a, out_hbm.at[pl.ds(ebase + j * CH, CH)])

    return k(pa, pb, src, dst)


def _sc_scatter_s(cw, c2e_edge, c2e_cycle, n_edges):
    D = cw.shape[1]
    M = c2e_edge.shape[0]
    RS = 12032
    ACC_ROWS = RS + 128
    NW = -(-n_edges // RS)
    NPASS = -(-NW // _NC)
    MPT = M // _NS
    CH = 64
    NCH = MPT // CH
    ZPT = ACC_ROWS // _NS

    @functools.partial(
        pl.kernel,
        out_type=jax.ShapeDtypeStruct((n_edges, D), jnp.float32),
        mesh=_mesh(),
        scratch_types=[
            pltpu.VMEM_SHARED((ACC_ROWS, D), jnp.float32),
            pltpu.VMEM((MPT,), jnp.int32),
            pltpu.VMEM((MPT,), jnp.int32),
            pltpu.VMEM((NCH, CH), jnp.int32),
            pltpu.VMEM((CH, D), jnp.float32),
            pltpu.VMEM((CH, D), jnp.float32),
            pltpu.SemaphoreType.DMA,
            pltpu.SemaphoreType.DMA,
        ],
    )
    def k(cw_hbm, ce_hbm, cc_hbm, out_hbm, acc, ebuf, cbuf, li,
          rows_a, rows_b, sem_a, sem_b):
        c = lax.axis_index("c")
        s = lax.axis_index("s")
        mbase = s * MPT
        pltpu.sync_copy(ce_hbm.at[pl.ds(mbase, MPT)], ebuf)
        pltpu.sync_copy(cc_hbm.at[pl.ds(mbase, MPT)], cbuf)
        def zero_rows_a():
            _zero_zbuf(rows_a)
        for p in range(NPASS):
            base = (_NC * p + c) * RS
            zero_rows_a()
            _zero_acc_range(acc, rows_a, s * ZPT, ZPT)
            def cli(jr, _):
                for u in range(CH // _L):
                    e = ebuf[pl.ds(jr * CH + u * _L, _L)]
                    inr = (e >= base) & (e < base + RS)
                    li[jr, pl.ds(u * _L, _L)] = jnp.where(inr, e - base,
                                                          RS + s * 8)
                return 0
            lax.fori_loop(0, NCH, cli, 0)
            plsc.subcore_barrier()
            def sct(j2, _):
                j = 2 * j2
                ga = pltpu.async_copy(cw_hbm.at[cbuf.at[pl.ds(j * CH, CH)]],
                                      rows_a, sem_a)
                gb = pltpu.async_copy(
                    cw_hbm.at[cbuf.at[pl.ds((j + 1) * CH, CH)]], rows_b, sem_b)
                ga.wait()
                pltpu.sync_copy(rows_a, acc.at[li.at[j]], add=True)
                gb.wait()
                pltpu.sync_copy(rows_b, acc.at[li.at[j + 1]], add=True)
                return 0
            lax.fori_loop(0, NCH // 2, sct, 0)
            plsc.subcore_barrier()
            for cc in range(_NC):
                w = _NC * p + cc
                if w >= NW:
                    continue
                wbase = w * RS
                wrows = min(RS, n_edges - wbase)
                dpt = wrows // _NS
                @pl.when(c == cc)
                def _():
                    pltpu.sync_copy(
                        acc.at[pl.ds(s * dpt, dpt)],
                        out_hbm.at[pl.ds(wbase + s * dpt, dpt)])
            plsc.subcore_barrier()

    return k(cw, c2e_edge, c2e_cycle)


def _mm_body(x_ref, w_ref, out_ref):
    out_ref[...] = jnp.dot(x_ref[...], w_ref[...],
                           preferred_element_type=jnp.float32
                           ).astype(out_ref.dtype)


def _mm(x, w, block, out_dtype=jnp.float32):
    n, d = x.shape
    d2 = w.shape[1]
    return pl.pallas_call(
        _mm_body,
        grid=(n // block,),
        in_specs=[pl.BlockSpec((block, d), lambda i: (i, 0)),
                  pl.BlockSpec((d, d2), lambda i: (0, 0))],
        out_specs=pl.BlockSpec((block, d2), lambda i: (i, 0)),
        out_shape=jax.ShapeDtypeStruct((n, d2), out_dtype),
    )(x, w)


def _node_body(x_ref, a0_ref, a1_ref, w0_ref, w1_ref, w2_ref, out_ref):
    h = x_ref[...] + jnp.dot(a0_ref[...] + a1_ref[...], w0_ref[...],
                             preferred_element_type=jnp.float32)
    h = jnp.maximum(jnp.dot(h, w1_ref[...],
                            preferred_element_type=jnp.float32), 0.0)
    out_ref[...] = jnp.dot(h, w2_ref[...], preferred_element_type=jnp.float32)


def _node_mlp(x, a0, a1, w0, w1, w2, block):
    n, d = x.shape
    bs = pl.BlockSpec((block, d), lambda i: (i, 0))
    ws = pl.BlockSpec((d, d), lambda i: (0, 0))
    return pl.pallas_call(
        _node_body,
        grid=(n // block,),
        in_specs=[bs, bs, bs, ws, ws, ws],
        out_specs=bs,
        out_shape=jax.ShapeDtypeStruct((n, d), jnp.float32),
    )(x, a0, a1, w0, w1, w2)


def _gin_body(x_ref, a0_ref, a1_ref, w1_ref, w2_ref, out_ref):
    h = x_ref[...] + a0_ref[...] + a1_ref[...]
    h = jnp.maximum(jnp.dot(h, w1_ref[...],
                            preferred_element_type=jnp.float32), 0.0)
    out_ref[...] = jnp.dot(h, w2_ref[...], preferred_element_type=jnp.float32)


def _gin(x, a0, a1, w1, w2, block):
    n, d = x.shape
    bs = pl.BlockSpec((block, d), lambda i: (i, 0))
    ws = pl.BlockSpec((d, d), lambda i: (0, 0))
    return pl.pallas_call(
        _gin_body,
        grid=(n // block,),
        in_specs=[bs, bs, bs, ws, ws],
        out_specs=bs,
        out_shape=jax.ShapeDtypeStruct((n, d), jnp.float32),
    )(x, a0, a1, w1, w2)


def _edge_body(edge_ref, a1_ref, s_ref, wec1_ref, wec2_ref, wm1a_ref,
               wm1b_ref, wm2_ref, out_ref):
    e1 = jnp.maximum(a1_ref[...], 0.0)
    t = jnp.dot(edge_ref[...], wec1_ref[...],
                preferred_element_type=jnp.float32) + s_ref[...].astype(jnp.float32)
    e2 = jnp.dot(jnp.maximum(t, 0.0), wec2_ref[...],
                 preferred_element_type=jnp.float32)
    m = jnp.dot(e1, wm1a_ref[...], preferred_element_type=jnp.float32)
    m += jnp.dot(e2, wm1b_ref[...], preferred_element_type=jnp.float32)
    out_ref[...] = jnp.dot(jnp.maximum(m, 0.0), wm2_ref[...],
                           preferred_element_type=jnp.float32)


def _edge_mlp(edge_rep, a1, s, wec1s, wec2, wm1a, wm1b, wm2, block):
    e, d = edge_rep.shape
    bs = pl.BlockSpec((block, d), lambda i: (i, 0))
    ws = pl.BlockSpec((d, d), lambda i: (0, 0))
    return pl.pallas_call(
        _edge_body,
        grid=(e // block,),
        in_specs=[bs, bs, bs, ws, ws, ws, ws, ws],
        out_specs=bs,
        out_shape=jax.ShapeDtypeStruct((e, d), jnp.float32),
    )(edge_rep, a1, s, wec1s, wec2, wm1a, wm1b, wm2)


def kernel(node_rep, edge_rep, cycle_rep, edge_index, c2e_edge, c2e_cycle,
           W_e2n, W_n1, W_n2, W_ne1, W_ec1, W_ec2, W_ce1, W_ce2, W_m1, W_m2,
           eps_e, eps_c):
    n_nodes, d = node_rep.shape
    n_edges = edge_rep.shape[0]
    n_cycles = cycle_rep.shape[0]
    src = edge_index[0]
    dst = edge_index[1]

    W_a, W_b = W_ne1[:d], W_ne1[d:]
    pa = _mm(node_rep, W_a, block=1000)
    pb = _mm(node_rep, W_b, block=1000)
    cw = _mm(cycle_rep, W_ec1, block=1024)

    aggn2, npad = _sc_agg_n(edge_rep, dst, n_nodes)
    aggc2 = _sc_agg_c(edge_rep, c2e_edge, c2e_cycle, n_cycles)
    a1 = _sc_a1(pa, pb, src, dst, n_edges)
    s = _sc_scatter_s(cw, c2e_edge, c2e_cycle, n_edges)

    node_out = _node_mlp(node_rep, aggn2[:n_nodes], aggn2[npad:npad + n_nodes],
                         W_e2n, W_n1, W_n2, block=1000)
    cin = (1.0 + eps_c) * cycle_rep
    cycle_out = _gin(cin, aggc2[:n_cycles], aggc2[n_cycles:],
                     W_ce1, W_ce2, block=1024)
    wec1s = (1.0 + eps_e) * W_ec1
    W_m1a, W_m1b = W_m1[:d], W_m1[d:]
    edge_out = _edge_mlp(edge_rep, a1, s, wec1s, W_ec2, W_m1a, W_m1b, W_m2,
                         block=2000)
    return (node_out, edge_out, cycle_out)
